# baseline (speedup 1.0000x reference)
import functools

import jax
import jax.numpy as jnp
from jax import lax
from jax.experimental import pallas as pl
from jax.experimental.pallas import tpu as pltpu
from jax.experimental.pallas import tpu_sc as plsc

_C = 128


def _sc_label_gather(lab, typ, proc, med, chart):
    N = lab.shape[0]
    H = proc.shape[1]
    mesh = plsc.VectorSubcoreMesh(
        core_axis_name="c", subcore_axis_name="s", num_cores=2, num_subcores=16
    )
    nw = 32
    per_w = N // nw
    n_chunks = per_w // _C

    @functools.partial(
        pl.kernel,
        out_type=jax.ShapeDtypeStruct((N, H), jnp.float32),
        mesh=mesh,
        compiler_params=pltpu.CompilerParams(use_tc_tiling_on_sc=False),
        scratch_types=[
            pltpu.VMEM((_C,), jnp.int32),
            pltpu.VMEM((_C,), jnp.int32),
            pltpu.VMEM((_C,), jnp.int32),
            pltpu.VMEM((_C,), jnp.int32),
            pltpu.VMEM((_C,), jnp.int32),
            pltpu.VMEM((_C, H), jnp.float32),
            pltpu.SemaphoreType.DMA,
            pltpu.SemaphoreType.DMA,
        ],
    )
    def k(lab_hbm, typ_hbm, proc_hbm, med_hbm, chart_hbm, out_hbm,
          lab_v, typ_v, i1, i2, i3, rows, sem_a, sem_b):
        wid = lax.axis_index("s") * 2 + lax.axis_index("c")
        base = wid * per_w

        def body(g, carry):
            off = base + g * _C
            pltpu.sync_copy(lab_hbm.at[pl.ds(off, _C)], lab_v)
            pltpu.sync_copy(typ_hbm.at[pl.ds(off, _C)], typ_v)
            for r in range(_C // 16):
                sl = pl.ds(r * 16, 16)
                t = typ_v[sl]
                l = lab_v[sl]
                z = jnp.zeros((16,), jnp.int32)
                i1[sl] = jnp.where(t == 1, l, z)
                i2[sl] = jnp.where(t == 2, l, z)
                i3[sl] = jnp.where(t == 3, l, z)
            pltpu.async_copy(proc_hbm.at[i1], rows, sem_a).wait()
            pltpu.async_copy(med_hbm.at[i2], rows, sem_b, add=True).wait()
            pltpu.async_copy(chart_hbm.at[i3], rows, sem_b, add=True).wait()
            pltpu.sync_copy(rows, out_hbm.at[pl.ds(off, _C)])
            return carry

        lax.fori_loop(0, n_chunks, body, 0)

    return k(lab, typ, proc, med, chart)


def _tc_dense(gath, vals, times, typs, poss, ons, age_i, gen_i,
              D, W1, b1, W2, b2, lw, lb, pw, pb,
              on_tab, pos_tab, age_tab, gen_tab, task_tab, g_, beta_):
    B, S, H = gath.shape
    BB = 8
    G = B // BB
    P = pos_tab.shape[0]
    OV = on_tab.shape[0]
    NA = age_tab.shape[0]
    M = BB * S

    def body(gath_ref, val_ref, time_ref, typ_ref, pos_ref, on_ref,
             age_ref, gen_ref, D_ref, W1_ref, b1_ref, W2_ref, b2_ref,
             lw_ref, lb_ref, pw_ref, pb_ref, ontab_ref, postab_ref,
             agetab_ref, gentab_ref, tasktab_ref, g_ref, beta_ref, out_ref):
        rows = gath_ref[...].reshape(M, H)
        t = typ_ref[...].reshape(M, 1)
        d0 = D_ref[0:1, :]
        d1 = D_ref[1:2, :]
        d2 = D_ref[2:3, :]
        d3 = D_ref[3:4, :]
        corr = jnp.where(t == 1, d1, jnp.where(t == 2, d2, jnp.where(t == 3, d3, d0)))
        v = val_ref[...].reshape(M, 1)
        h1 = jnp.maximum(v * W1_ref[...] + b1_ref[...], 0.0)
        ve = jnp.dot(h1, W2_ref[...], preferred_element_type=jnp.float32) + b2_ref[...]
        tt = time_ref[...].reshape(M, 1)
        lane = lax.broadcasted_iota(jnp.int32, (M, H), 1)
        te = jnp.where(lane == 0, tt * lw_ref[...] + lb_ref[...],
                       jnp.sin(tt * pw_ref[...] + pb_ref[...]))
        pid = pos_ref[...].reshape(M, 1)
        ph = (lax.broadcasted_iota(jnp.int32, (M, P), 1) == pid).astype(jnp.float32)
        pe = jnp.dot(ph, postab_ref[...], preferred_element_type=jnp.float32)
        oid = on_ref[...].reshape(M, 1)
        oh = (lax.broadcasted_iota(jnp.int32, (M, OV), 1) == oid).astype(jnp.float32)
        oe = jnp.dot(oh, ontab_ref[...], preferred_element_type=jnp.float32)
        emb = rows + corr + ve + te + pe + oe

        def norm(x):
            mu = jnp.mean(x, axis=-1, keepdims=True)
            var = jnp.mean((x - mu) ** 2, axis=-1, keepdims=True)
            return (x - mu) / jnp.sqrt(var + 1e-12) * g_ref[...] + beta_ref[...]

        out_ref[:, 3:, :] = norm(emb).reshape(BB, S, H)
        task_n = norm(tasktab_ref[...]).reshape(1, 1, H)
        out_ref[:, 0:1, :] = jnp.broadcast_to(task_n, (BB, 1, H))
        arows = []
        grows = []
        for r in range(BB):
            aid = age_ref[r, 0]
            am = lax.broadcasted_iota(jnp.int32, (NA, H), 0) == aid
            arows.append(jnp.sum(jnp.where(am, agetab_ref[...], 0.0), axis=0,
                                 keepdims=True))
            gid = gen_ref[r, 0]
            gm = lax.broadcasted_iota(jnp.int32, (2, H), 0) == gid
            grows.append(jnp.sum(jnp.where(gm, gentab_ref[...], 0.0), axis=0,
                                 keepdims=True))
        ae = jnp.concatenate(arows, axis=0)
        ge = jnp.concatenate(grows, axis=0)
        out_ref[:, 1:2, :] = norm(ae).reshape(BB, 1, H)
        out_ref[:, 2:3, :] = norm(ge).reshape(BB, 1, H)

    const = lambda shape: pl.BlockSpec(shape, lambda i: tuple(0 for _ in shape))
    return pl.pallas_call(
        body,
        grid=(G,),
        in_specs=[
            pl.BlockSpec((BB, S, H), lambda i: (i, 0, 0)),
            pl.BlockSpec((BB, S, 1), lambda i: (i, 0, 0)),
            pl.BlockSpec((BB, S, 1), lambda i: (i, 0, 0)),
            pl.BlockSpec((BB, S, 1), lambda i: (i, 0, 0)),
            pl.BlockSpec((BB, S, 1), lambda i: (i, 0, 0)),
            pl.BlockSpec((BB, S, 1), lambda i: (i, 0, 0)),
            pl.BlockSpec((BB, 1), lambda i: (i, 0), memory_space=pltpu.SMEM),
            pl.BlockSpec((BB, 1), lambda i: (i, 0), memory_space=pltpu.SMEM),
            const((4, H)),
            const((1, H)),
            const((1, H)),
            const((H, H)),
            const((1, H)),
            const((1, 1)),
            const((1, 1)),
            const((1, H)),
            const((1, H)),
            const((OV, H)),
            const((P, H)),
            const((NA, H)),
            const((2, H)),
            const((1, H)),
            const((1, H)),
            const((1, H)),
        ],
        out_specs=pl.BlockSpec((BB, 3 + S, H), lambda i: (i, 0, 0)),
        out_shape=jax.ShapeDtypeStruct((B, 3 + S, H), jnp.float32),
    )(gath, vals, times, typs, poss, ons, age_i, gen_i, D, W1, b1, W2, b2,
      lw, lb, pw, pb, on_tab, pos_tab, age_tab, gen_tab, task_tab, g_, beta_)


def kernel(label_ids, value_ids, time_ids, on_ids, position_ids, token_type,
           age_ids, gender_ids, task_token, proc_table, med_table, chart_table,
           W1, b1, W2, b2, t2v_lw, t2v_lb, t2v_pw, t2v_pb,
           on_table, pos_table, age_table, gender_table, task_table, ln_g, ln_b):
    B, S = label_ids.shape
    H = proc_table.shape[1]
    N = B * S
    lab = label_ids.reshape(N).astype(jnp.int32)
    typ = token_type.reshape(N).astype(jnp.int32)
    gath = _sc_label_gather(lab, typ, proc_table, med_table, chart_table)
    gath = gath.reshape(B, S, H)
    p0 = proc_table[0]
    m0 = med_table[0]
    c0 = chart_table[0]
    R = p0 + m0 + c0
    D = jnp.stack([-R, p0 - R, m0 - R, c0 - R], axis=0)
    pw_pad = jnp.concatenate([jnp.zeros((1, 1), jnp.float32), t2v_pw], axis=1)
    pb_pad = jnp.concatenate([jnp.zeros((1,), jnp.float32), t2v_pb]).reshape(1, H)
    on_pad = jnp.concatenate(
        [on_table, jnp.zeros((16 - on_table.shape[0], H), jnp.float32)], axis=0)
    return _tc_dense(
        gath,
        value_ids.reshape(B, S, 1),
        time_ids.reshape(B, S, 1),
        token_type.reshape(B, S, 1).astype(jnp.int32),
        position_ids.reshape(B, S, 1).astype(jnp.int32),
        on_ids.reshape(B, S, 1).astype(jnp.int32),
        age_ids.astype(jnp.int32),
        gender_ids.astype(jnp.int32),
        D, W1, b1.reshape(1, H), W2, b2.reshape(1, H),
        t2v_lw, t2v_lb.reshape(1, 1), pw_pad, pb_pad,
        on_pad, pos_table, age_table, gender_table, task_table,
        ln_g.reshape(1, H), ln_b.reshape(1, H))


# speedup vs baseline: 1.6198x; 1.6198x over previous
import functools

import numpy as np
import jax
import jax.numpy as jnp
from jax import lax
from jax.experimental import pallas as pl
from jax.experimental.pallas import tpu as pltpu
from jax.experimental.pallas import tpu_sc as plsc

_C = 128
_R = 5


def _sc_label_gather(lab, typ, stk, V):
    N = lab.shape[0]
    H = stk.shape[1]
    zidx = 3 * V
    mesh = plsc.VectorSubcoreMesh(
        core_axis_name="c", subcore_axis_name="s", num_cores=2, num_subcores=16
    )
    nw = 32
    per_w = N // nw
    n_chunks = per_w // _C
    n_waves = n_chunks // _R

    @functools.partial(
        pl.kernel,
        out_type=jax.ShapeDtypeStruct((N, H), jnp.float32),
        mesh=mesh,
        compiler_params=pltpu.CompilerParams(use_tc_tiling_on_sc=False),
        scratch_types=[
            pltpu.VMEM((per_w,), jnp.int32),
            pltpu.VMEM((per_w,), jnp.int32),
        ]
        + [pltpu.VMEM((_C,), jnp.int32) for _ in range(_R)]
        + [pltpu.VMEM((_C, H), jnp.float32) for _ in range(_R)]
        + [pltpu.SemaphoreType.DMA, pltpu.SemaphoreType.DMA],
    )
    def k(lab_hbm, typ_hbm, stk_hbm, out_hbm, lab_v, typ_v, *rest):
        idx_bufs = rest[:_R]
        row_bufs = rest[_R:2 * _R]
        gsem, wsem = rest[2 * _R], rest[2 * _R + 1]
        wid = lax.axis_index("s") * 2 + lax.axis_index("c")
        base = wid * per_w
        pltpu.sync_copy(lab_hbm.at[pl.ds(base, per_w)], lab_v)
        pltpu.sync_copy(typ_hbm.at[pl.ds(base, per_w)], typ_v)

        def wave(w, carry):
            g0 = w * _R
            gathers = []
            for b in range(_R):
                tok = (g0 + b) * _C
                for r in range(_C // 16):
                    sl_src = pl.ds(tok + r * 16, 16)
                    sl_dst = pl.ds(r * 16, 16)
                    t = typ_v[sl_src]
                    l = lab_v[sl_src]
                    z = jnp.full((16,), zidx, jnp.int32)
                    idx_bufs[b][sl_dst] = jnp.where(t == 0, z, l + (t - 1) * V)
                gathers.append(
                    pltpu.async_copy(stk_hbm.at[idx_bufs[b]], row_bufs[b], gsem))
            for cp in gathers:
                cp.wait()
            writes = []
            for b in range(_R):
                tok = (g0 + b) * _C
                writes.append(
                    pltpu.async_copy(row_bufs[b], out_hbm.at[pl.ds(base + tok, _C)],
                                     wsem))
            for cp in writes:
                cp.wait()
            return carry

        lax.fori_loop(0, n_waves, wave, 0)

    return k(lab, typ, stk)


def _tc_dense(gath, vals, times, poss, ons, age_i, gen_i,
              W1, b1, W2, b2, lw, lb, pw, pb, posg, posp,
              on_tab, age_tab, gen_tab, task_tab, g_, beta_):
    B, S, H = gath.shape
    BB = 8
    G = B // BB
    OV = on_tab.shape[0]
    NA = age_tab.shape[0]
    M = BB * S

    def body(gath_ref, val_ref, time_ref, pos_ref, on_ref,
             age_ref, gen_ref, W1_ref, b1_ref, W2_ref, b2_ref,
             lw_ref, lb_ref, pw_ref, pb_ref, posg_ref, posp_ref,
             ontab_ref, agetab_ref, gentab_ref, tasktab_ref,
             g_ref, beta_ref, out_ref):
        rows = gath_ref[...].reshape(M, H)
        v = val_ref[...].reshape(M, 1)
        h1 = jnp.maximum(v * W1_ref[...] + b1_ref[...], 0.0)
        ve = jnp.dot(h1, W2_ref[...], preferred_element_type=jnp.float32) + b2_ref[...]
        tt = time_ref[...].reshape(M, 1)
        lane = lax.broadcasted_iota(jnp.int32, (M, H), 1)
        te = jnp.where(lane == 0, tt * lw_ref[...] + lb_ref[...],
                       jnp.sin(tt * pw_ref[...] + pb_ref[...]))
        pf = pos_ref[...].reshape(M, 1).astype(jnp.float32)
        pe = jnp.sin(pf * posg_ref[...] + posp_ref[...])
        oid = on_ref[...].reshape(M, 1)
        oh = (lax.broadcasted_iota(jnp.int32, (M, OV), 1) == oid).astype(jnp.float32)
        oe = jnp.dot(oh, ontab_ref[...], preferred_element_type=jnp.float32)
        emb = rows + ve + te + pe + oe

        def norm(x):
            mu = jnp.mean(x, axis=-1, keepdims=True)
            var = jnp.mean((x - mu) ** 2, axis=-1, keepdims=True)
            return (x - mu) / jnp.sqrt(var + 1e-12) * g_ref[...] + beta_ref[...]

        out_ref[:, 3:, :] = norm(emb).reshape(BB, S, H)
        task_n = norm(tasktab_ref[...]).reshape(1, 1, H)
        out_ref[:, 0:1, :] = jnp.broadcast_to(task_n, (BB, 1, H))
        arows = []
        grows = []
        for r in range(BB):
            aid = age_ref[r, 0]
            am = lax.broadcasted_iota(jnp.int32, (NA, H), 0) == aid
            arows.append(jnp.sum(jnp.where(am, agetab_ref[...], 0.0), axis=0,
                                 keepdims=True))
            gid = gen_ref[r, 0]
            gm = lax.broadcasted_iota(jnp.int32, (2, H), 0) == gid
            grows.append(jnp.sum(jnp.where(gm, gentab_ref[...], 0.0), axis=0,
                                 keepdims=True))
        ae = jnp.concatenate(arows, axis=0)
        ge = jnp.concatenate(grows, axis=0)
        out_ref[:, 1:2, :] = norm(ae).reshape(BB, 1, H)
        out_ref[:, 2:3, :] = norm(ge).reshape(BB, 1, H)

    const = lambda shape: pl.BlockSpec(shape, lambda i: tuple(0 for _ in shape))
    return pl.pallas_call(
        body,
        grid=(G,),
        in_specs=[
            pl.BlockSpec((BB, S, H), lambda i: (i, 0, 0)),
            pl.BlockSpec((BB, S, 1), lambda i: (i, 0, 0)),
            pl.BlockSpec((BB, S, 1), lambda i: (i, 0, 0)),
            pl.BlockSpec((BB, S, 1), lambda i: (i, 0, 0)),
            pl.BlockSpec((BB, S, 1), lambda i: (i, 0, 0)),
            pl.BlockSpec((BB, 1), lambda i: (i, 0), memory_space=pltpu.SMEM),
            pl.BlockSpec((BB, 1), lambda i: (i, 0), memory_space=pltpu.SMEM),
            const((1, H)),
            const((1, H)),
            const((H, H)),
            const((1, H)),
            const((1, 1)),
            const((1, 1)),
            const((1, H)),
            const((1, H)),
            const((1, H)),
            const((1, H)),
            const((OV, H)),
            const((NA, H)),
            const((2, H)),
            const((1, H)),
            const((1, H)),
            const((1, H)),
        ],
        out_specs=pl.BlockSpec((BB, 3 + S, H), lambda i: (i, 0, 0)),
        out_shape=jax.ShapeDtypeStruct((B, 3 + S, H), jnp.float32),
    )(gath, vals, times, poss, ons, age_i, gen_i, W1, b1, W2, b2,
      lw, lb, pw, pb, posg, posp, on_tab, age_tab, gen_tab, task_tab, g_, beta_)


def kernel(label_ids, value_ids, time_ids, on_ids, position_ids, token_type,
           age_ids, gender_ids, task_token, proc_table, med_table, chart_table,
           W1, b1, W2, b2, t2v_lw, t2v_lb, t2v_pw, t2v_pb,
           on_table, pos_table, age_table, gender_table, task_table, ln_g, ln_b):
    B, S = label_ids.shape
    H = proc_table.shape[1]
    V = proc_table.shape[0]
    N = B * S
    lab = label_ids.reshape(N).astype(jnp.int32)
    typ = token_type.reshape(N).astype(jnp.int32)
    stk = jnp.concatenate(
        [proc_table, med_table, chart_table, jnp.zeros((8, H), jnp.float32)], axis=0)
    gath = _sc_label_gather(lab, typ, stk, V).reshape(B, S, H)
    pw_pad = jnp.concatenate([jnp.zeros((1, 1), jnp.float32), t2v_pw], axis=1)
    pb_pad = jnp.concatenate([jnp.zeros((1,), jnp.float32), t2v_pb]).reshape(1, H)
    on_pad = jnp.concatenate(
        [on_table, jnp.zeros((16 - on_table.shape[0], H), jnp.float32)], axis=0)
    j = np.arange(H, dtype=np.float64)
    posg = jnp.asarray((10000.0 ** (-2.0 * j / H)).astype(np.float32)).reshape(1, H)
    posp = jnp.asarray(
        np.where(j % 2 == 0, 0.0, np.pi / 2).astype(np.float32)).reshape(1, H)
    return _tc_dense(
        gath,
        value_ids.reshape(B, S, 1),
        time_ids.reshape(B, S, 1),
        position_ids.reshape(B, S, 1).astype(jnp.int32),
        on_ids.reshape(B, S, 1).astype(jnp.int32),
        age_ids.astype(jnp.int32),
        gender_ids.astype(jnp.int32),
        W1, b1.reshape(1, H), W2, b2.reshape(1, H),
        t2v_lw, t2v_lb.reshape(1, 1), pw_pad, pb_pad, posg, posp,
        on_pad, age_table, gender_table, task_table,
        ln_g.reshape(1, H), ln_b.reshape(1, H))
